# baseline (speedup 1.0000x reference)
import jax
import jax.numpy as jnp
from jax.experimental import pallas as pl
from jax.experimental.pallas import tpu as pltpu

_B, _C, _H, _W = 64, 512, 28, 28
_HW = _H * _W
_D = 512
_GR = 512
_K = 20
_NC = 10
_EPS = 1e-5

_GRAM_BB = 8
_RK = 32
_RROWS = _C // _RK


def _gram_body(x_ref, g_ref):
    for b in range(_GRAM_BB):
        x = x_ref[b]
        g = jax.lax.dot_general(x, x, (((1,), (1,)), ((), ())),
                                preferred_element_type=jnp.float32)
        g_ref[b] = (g * (1.0 / _HW)).astype(jnp.bfloat16)


def _reduce_head_body(g_ref, w_ref, xe_ref, gb_ref, cent_ref,
                      aw1_ref, ab1_ref, aw2_ref, ab2_ref,
                      lfg_ref, lfb_ref, fw1_ref, fb1_ref, fw2_ref, fb2_ref,
                      gw_ref, gmb_ref, l1g_ref, l1b_ref, m1w_ref, m1b_ref,
                      m2w1_ref, m2b1_ref, m2w2_ref, m2b2_ref, ow_ref, ob_ref,
                      out_ref, acc_ref):
    k = pl.program_id(0)

    @pl.when(k == 0)
    def _():
        acc_ref[...] = jnp.zeros_like(acc_ref)

    acc = acc_ref[...]
    for j in range(_RROWS):
        acc = acc + jnp.dot(g_ref[:, j, :].astype(jnp.float32), w_ref[j],
                            preferred_element_type=jnp.float32)
    acc_ref[...] = acc

    @pl.when(k == _RK - 1)
    def _():
        _head(acc_ref[...], xe_ref, gb_ref, cent_ref,
              aw1_ref, ab1_ref, aw2_ref, ab2_ref,
              lfg_ref, lfb_ref, fw1_ref, fb1_ref, fw2_ref, fb2_ref,
              gw_ref, gmb_ref, l1g_ref, l1b_ref, m1w_ref, m1b_ref,
              m2w1_ref, m2b1_ref, m2w2_ref, m2b2_ref, ow_ref, ob_ref,
              out_ref)


def _gelu_exact(x):
    return x * 0.5 * (1.0 + jax.lax.erf(x * (2.0 ** -0.5)))


def _layernorm(x, g, b):
    m = jnp.mean(x, axis=-1, keepdims=True)
    v = jnp.mean((x - m) * (x - m), axis=-1, keepdims=True)
    return (x - m) / jnp.sqrt(v + _EPS) * g + b


def _head(part, xe_ref, gb_ref, cent_ref,
          aw1_ref, ab1_ref, aw2_ref, ab2_ref,
          lfg_ref, lfb_ref, fw1_ref, fb1_ref, fw2_ref, fb2_ref,
          gw_ref, gmb_ref, l1g_ref, l1b_ref, m1w_ref, m1b_ref,
          m2w1_ref, m2b1_ref, m2w2_ref, m2b2_ref, ow_ref, ob_ref,
          out_ref):
    x = xe_ref[...]
    a = jnp.dot(x, aw1_ref[...], preferred_element_type=jnp.float32) + ab1_ref[...]
    a = jnp.maximum(a, 0.0)
    a = jnp.dot(a, aw2_ref[...], preferred_element_type=jnp.float32) + ab2_ref[...]
    a = jax.nn.sigmoid(a)
    x_attn = x * a

    h = _layernorm(x_attn, lfg_ref[...], lfb_ref[...])
    h = jnp.dot(h, fw1_ref[...], preferred_element_type=jnp.float32) + fb1_ref[...]
    h = _gelu_exact(h)
    h = jnp.dot(h, fw2_ref[...], preferred_element_type=jnp.float32) + fb2_ref[...]
    x_main = x_attn + h

    gf = part + gb_ref[...]
    cents = cent_ref[...]
    x2 = jnp.sum(gf * gf, axis=1, keepdims=True)
    c2 = jnp.sum(cents * cents, axis=1)[None, :]
    cross = jax.lax.dot_general(gf, cents, (((1,), (1,)), ((), ())),
                                preferred_element_type=jnp.float32)
    d2 = x2 - 2.0 * cross + c2
    dmin = jnp.min(d2, axis=1, keepdims=True)
    m = (d2 <= dmin).astype(jnp.float32)
    ii = jax.lax.broadcasted_iota(jnp.int32, (_K, _K), 0)
    jj = jax.lax.broadcasted_iota(jnp.int32, (_K, _K), 1)
    lstrict = (ii < jj).astype(jnp.float32)
    prior = jnp.dot(m, lstrict, preferred_element_type=jnp.float32)
    onehot = jnp.where(prior == 0.0, m, 0.0)
    cluster = jnp.dot(onehot, cents, preferred_element_type=jnp.float32)

    xh = (jnp.dot(x_main, gw_ref[: _D], preferred_element_type=jnp.float32)
          + jnp.dot(cluster, gw_ref[_D:], preferred_element_type=jnp.float32)
          + gmb_ref[...])
    xh = _gelu_exact(xh)

    h = _layernorm(xh, l1g_ref[...], l1b_ref[...])
    h = jnp.dot(h, m1w_ref[...], preferred_element_type=jnp.float32) + m1b_ref[...]
    h = _gelu_exact(h)
    h = h + xh
    h = jnp.dot(h, m2w1_ref[...], preferred_element_type=jnp.float32) + m2b1_ref[...]
    h = _gelu_exact(h)
    h = jnp.dot(h, m2w2_ref[...], preferred_element_type=jnp.float32) + m2b2_ref[...]
    h = _gelu_exact(h)
    out_ref[...] = (jnp.dot(h, ow_ref[...], preferred_element_type=jnp.float32)
                    + ob_ref[...])


def kernel(x_embed, x_image, attn_w1, attn_b1, attn_w2, attn_b2,
           ln_ff_g, ln_ff_b, ff_w1, ff_b1, ff_w2, ff_b2,
           gram_w, gram_b, centers, gm_w, gm_b,
           ln1_g, ln1_b, m1_w, m1_b,
           m2_w1, m2_b1, m2_w2, m2_b2, out_w, out_b):
    xf = x_image.astype(jnp.bfloat16).reshape(_B, _C, _HW)

    gram = pl.pallas_call(
        _gram_body,
        out_shape=jax.ShapeDtypeStruct((_B, _C, _C), jnp.bfloat16),
        grid=(_B // _GRAM_BB,),
        in_specs=[pl.BlockSpec((_GRAM_BB, _C, _HW), lambda i: (i, 0, 0))],
        out_specs=pl.BlockSpec((_GRAM_BB, _C, _C), lambda i: (i, 0, 0)),
        compiler_params=pltpu.CompilerParams(
            dimension_semantics=("parallel",),
            vmem_limit_bytes=50 * 1024 * 1024,
        ),
        name="gram_bmm",
    )(xf)

    w3 = gram_w.reshape(_C, _C, _GR)

    full = lambda x: pl.BlockSpec(x.shape, lambda k: tuple(0 for _ in x.shape))
    head_args = (x_embed, gram_b, centers,
                 attn_w1, attn_b1, attn_w2, attn_b2,
                 ln_ff_g, ln_ff_b, ff_w1, ff_b1, ff_w2, ff_b2,
                 gm_w, gm_b, ln1_g, ln1_b, m1_w, m1_b,
                 m2_w1, m2_b1, m2_w2, m2_b2, out_w, out_b)
    logits = pl.pallas_call(
        _reduce_head_body,
        out_shape=jax.ShapeDtypeStruct((_B, _NC), jnp.float32),
        grid=(_RK,),
        in_specs=[
            pl.BlockSpec((_B, _RROWS, _C), lambda k: (0, k, 0)),
            pl.BlockSpec((_RROWS, _C, _GR), lambda k: (k, 0, 0)),
        ] + [full(a) for a in head_args],
        out_specs=pl.BlockSpec((_B, _NC), lambda k: (0, 0)),
        scratch_shapes=[pltpu.VMEM((_B, _GR), jnp.float32)],
        compiler_params=pltpu.CompilerParams(
            dimension_semantics=("arbitrary",),
            vmem_limit_bytes=56 * 1024 * 1024,
        ),
        name="gram_reduce_head",
    )(gram, w3, *head_args)
    return logits


# speedup vs baseline: 1.0359x; 1.0359x over previous
import jax
import jax.numpy as jnp
from jax.experimental import pallas as pl
from jax.experimental.pallas import tpu as pltpu

_B, _C, _H, _W = 64, 512, 28, 28
_HW = _H * _W
_D = 512
_GR = 512
_K = 20
_NC = 10
_EPS = 1e-5

_GRAM_BB = 8
_RK = 32
_RROWS = _C // _RK


def _gram_body(x_ref, g_ref):
    for b in range(_GRAM_BB):
        x = x_ref[b]
        g = jax.lax.dot_general(x, x, (((1,), (1,)), ((), ())),
                                preferred_element_type=jnp.float32)
        g_ref[b] = (g * (1.0 / _HW)).astype(jnp.bfloat16)


def _reduce_head_body(g_ref, w_ref, xe_ref, gb_ref, cent_ref,
                      aw1_ref, ab1_ref, aw2_ref, ab2_ref,
                      lfg_ref, lfb_ref, fw1_ref, fb1_ref, fw2_ref, fb2_ref,
                      gw_ref, gmb_ref, l1g_ref, l1b_ref, m1w_ref, m1b_ref,
                      m2w1_ref, m2b1_ref, m2w2_ref, m2b2_ref, ow_ref, ob_ref,
                      out_ref, acc_ref):
    k = pl.program_id(0)

    @pl.when(k == 0)
    def _():
        acc_ref[...] = jnp.zeros_like(acc_ref)

    acc = acc_ref[...]
    for j in range(_RROWS):
        acc = acc + jnp.dot(g_ref[:, j, :].astype(jnp.float32), w_ref[j],
                            preferred_element_type=jnp.float32)
    acc_ref[...] = acc

    @pl.when(k == _RK - 1)
    def _():
        _head(acc_ref[...], xe_ref, gb_ref, cent_ref,
              aw1_ref, ab1_ref, aw2_ref, ab2_ref,
              lfg_ref, lfb_ref, fw1_ref, fb1_ref, fw2_ref, fb2_ref,
              gw_ref, gmb_ref, l1g_ref, l1b_ref, m1w_ref, m1b_ref,
              m2w1_ref, m2b1_ref, m2w2_ref, m2b2_ref, ow_ref, ob_ref,
              out_ref)


def _gelu_exact(x):
    return x * 0.5 * (1.0 + jax.lax.erf(x * (2.0 ** -0.5)))


def _layernorm(x, g, b):
    m = jnp.mean(x, axis=-1, keepdims=True)
    v = jnp.mean((x - m) * (x - m), axis=-1, keepdims=True)
    return (x - m) / jnp.sqrt(v + _EPS) * g + b


def _head(part, xe_ref, gb_ref, cent_ref,
          aw1_ref, ab1_ref, aw2_ref, ab2_ref,
          lfg_ref, lfb_ref, fw1_ref, fb1_ref, fw2_ref, fb2_ref,
          gw_ref, gmb_ref, l1g_ref, l1b_ref, m1w_ref, m1b_ref,
          m2w1_ref, m2b1_ref, m2w2_ref, m2b2_ref, ow_ref, ob_ref,
          out_ref):
    x = xe_ref[...]
    a = jnp.dot(x, aw1_ref[...], preferred_element_type=jnp.float32) + ab1_ref[...]
    a = jnp.maximum(a, 0.0)
    a = jnp.dot(a, aw2_ref[...], preferred_element_type=jnp.float32) + ab2_ref[...]
    a = jax.nn.sigmoid(a)
    x_attn = x * a

    h = _layernorm(x_attn, lfg_ref[...], lfb_ref[...])
    h = jnp.dot(h, fw1_ref[...], preferred_element_type=jnp.float32) + fb1_ref[...]
    h = _gelu_exact(h)
    h = jnp.dot(h, fw2_ref[...], preferred_element_type=jnp.float32) + fb2_ref[...]
    x_main = x_attn + h

    gf = part + gb_ref[...]
    cents = cent_ref[...]
    x2 = jnp.sum(gf * gf, axis=1, keepdims=True)
    c2 = jnp.sum(cents * cents, axis=1)[None, :]
    cross = jax.lax.dot_general(gf, cents, (((1,), (1,)), ((), ())),
                                preferred_element_type=jnp.float32)
    d2 = x2 - 2.0 * cross + c2
    dmin = jnp.min(d2, axis=1, keepdims=True)
    m = (d2 <= dmin).astype(jnp.float32)
    ii = jax.lax.broadcasted_iota(jnp.int32, (_K, _K), 0)
    jj = jax.lax.broadcasted_iota(jnp.int32, (_K, _K), 1)
    lstrict = (ii < jj).astype(jnp.float32)
    prior = jnp.dot(m, lstrict, preferred_element_type=jnp.float32)
    onehot = jnp.where(prior == 0.0, m, 0.0)
    cluster = jnp.dot(onehot, cents, preferred_element_type=jnp.float32)

    xh = (jnp.dot(x_main, gw_ref[: _D], preferred_element_type=jnp.float32)
          + jnp.dot(cluster, gw_ref[_D:], preferred_element_type=jnp.float32)
          + gmb_ref[...])
    xh = _gelu_exact(xh)

    h = _layernorm(xh, l1g_ref[...], l1b_ref[...])
    h = jnp.dot(h, m1w_ref[...], preferred_element_type=jnp.float32) + m1b_ref[...]
    h = _gelu_exact(h)
    h = h + xh
    h = jnp.dot(h, m2w1_ref[...], preferred_element_type=jnp.float32) + m2b1_ref[...]
    h = _gelu_exact(h)
    h = jnp.dot(h, m2w2_ref[...], preferred_element_type=jnp.float32) + m2b2_ref[...]
    h = _gelu_exact(h)
    out_ref[...] = (jnp.dot(h, ow_ref[...], preferred_element_type=jnp.float32)
                    + ob_ref[...])


def kernel(x_embed, x_image, attn_w1, attn_b1, attn_w2, attn_b2,
           ln_ff_g, ln_ff_b, ff_w1, ff_b1, ff_w2, ff_b2,
           gram_w, gram_b, centers, gm_w, gm_b,
           ln1_g, ln1_b, m1_w, m1_b,
           m2_w1, m2_b1, m2_w2, m2_b2, out_w, out_b):
    xf = x_image.reshape(_B, _C, _HW)

    gram = pl.pallas_call(
        _gram_body,
        out_shape=jax.ShapeDtypeStruct((_B, _C, _C), jnp.bfloat16),
        grid=(_B // _GRAM_BB,),
        in_specs=[pl.BlockSpec((_GRAM_BB, _C, _HW), lambda i: (i, 0, 0))],
        out_specs=pl.BlockSpec((_GRAM_BB, _C, _C), lambda i: (i, 0, 0)),
        compiler_params=pltpu.CompilerParams(
            dimension_semantics=("parallel",),
            vmem_limit_bytes=50 * 1024 * 1024,
        ),
        name="gram_bmm",
    )(xf)

    w3 = gram_w.reshape(_C, _C, _GR)

    full = lambda x: pl.BlockSpec(x.shape, lambda k: tuple(0 for _ in x.shape))
    head_args = (x_embed, gram_b, centers,
                 attn_w1, attn_b1, attn_w2, attn_b2,
                 ln_ff_g, ln_ff_b, ff_w1, ff_b1, ff_w2, ff_b2,
                 gm_w, gm_b, ln1_g, ln1_b, m1_w, m1_b,
                 m2_w1, m2_b1, m2_w2, m2_b2, out_w, out_b)
    logits = pl.pallas_call(
        _reduce_head_body,
        out_shape=jax.ShapeDtypeStruct((_B, _NC), jnp.float32),
        grid=(_RK,),
        in_specs=[
            pl.BlockSpec((_B, _RROWS, _C), lambda k: (0, k, 0)),
            pl.BlockSpec((_RROWS, _C, _GR), lambda k: (k, 0, 0)),
        ] + [full(a) for a in head_args],
        out_specs=pl.BlockSpec((_B, _NC), lambda k: (0, 0)),
        scratch_shapes=[pltpu.VMEM((_B, _GR), jnp.float32)],
        compiler_params=pltpu.CompilerParams(
            dimension_semantics=("arbitrary",),
            vmem_limit_bytes=56 * 1024 * 1024,
        ),
        name="gram_reduce_head",
    )(gram, w3, *head_args)
    return logits


# speedup vs baseline: 1.0395x; 1.0035x over previous
import jax
import jax.numpy as jnp
from jax.experimental import pallas as pl
from jax.experimental.pallas import tpu as pltpu

_B, _C, _H, _W = 64, 512, 28, 28
_HW = _H * _W
_D = 512
_GR = 512
_K = 20
_NC = 10
_EPS = 1e-5

_GRAM_BB = 8
_RK = 32
_RROWS = _C // _RK


def _gram_body(x_ref, g_ref):
    for b in range(_GRAM_BB):
        x = x_ref[b]
        g = jax.lax.dot_general(x, x, (((1,), (1,)), ((), ())),
                                preferred_element_type=jnp.float32)
        g_ref[b] = (g * (1.0 / _HW)).astype(jnp.bfloat16)


def _reduce_head_body(g_ref, w_ref, xe_ref, gb_ref, cent_ref,
                      aw1_ref, ab1_ref, aw2_ref, ab2_ref,
                      lfg_ref, lfb_ref, fw1_ref, fb1_ref, fw2_ref, fb2_ref,
                      gw_ref, gmb_ref, l1g_ref, l1b_ref, m1w_ref, m1b_ref,
                      m2w1_ref, m2b1_ref, m2w2_ref, m2b2_ref, ow_ref, ob_ref,
                      out_ref, acc_ref):
    k = pl.program_id(0)

    @pl.when(k == 0)
    def _():
        acc_ref[...] = jnp.zeros_like(acc_ref)

    acc = acc_ref[...]
    for j in range(_RROWS):
        acc = acc + jnp.dot(g_ref[:, j, :].astype(jnp.float32), w_ref[j],
                            preferred_element_type=jnp.float32)
    acc_ref[...] = acc

    @pl.when(k == _RK - 1)
    def _():
        _head(acc_ref[...], xe_ref, gb_ref, cent_ref,
              aw1_ref, ab1_ref, aw2_ref, ab2_ref,
              lfg_ref, lfb_ref, fw1_ref, fb1_ref, fw2_ref, fb2_ref,
              gw_ref, gmb_ref, l1g_ref, l1b_ref, m1w_ref, m1b_ref,
              m2w1_ref, m2b1_ref, m2w2_ref, m2b2_ref, ow_ref, ob_ref,
              out_ref)


def _gelu_exact(x):
    return x * 0.5 * (1.0 + jax.lax.erf(x * (2.0 ** -0.5)))


def _layernorm(x, g, b):
    m = jnp.mean(x, axis=-1, keepdims=True)
    v = jnp.mean((x - m) * (x - m), axis=-1, keepdims=True)
    return (x - m) / jnp.sqrt(v + _EPS) * g + b


def _head(part, xe_ref, gb_ref, cent_ref,
          aw1_ref, ab1_ref, aw2_ref, ab2_ref,
          lfg_ref, lfb_ref, fw1_ref, fb1_ref, fw2_ref, fb2_ref,
          gw_ref, gmb_ref, l1g_ref, l1b_ref, m1w_ref, m1b_ref,
          m2w1_ref, m2b1_ref, m2w2_ref, m2b2_ref, ow_ref, ob_ref,
          out_ref):
    x = xe_ref[...]
    a = jnp.dot(x, aw1_ref[...], preferred_element_type=jnp.float32) + ab1_ref[...]
    a = jnp.maximum(a, 0.0)
    a = jnp.dot(a, aw2_ref[...], preferred_element_type=jnp.float32) + ab2_ref[...]
    a = jax.nn.sigmoid(a)
    x_attn = x * a

    h = _layernorm(x_attn, lfg_ref[...], lfb_ref[...])
    h = jnp.dot(h, fw1_ref[...], preferred_element_type=jnp.float32) + fb1_ref[...]
    h = _gelu_exact(h)
    h = jnp.dot(h, fw2_ref[...], preferred_element_type=jnp.float32) + fb2_ref[...]
    x_main = x_attn + h

    gf = part + gb_ref[...]
    cents = cent_ref[...]
    x2 = jnp.sum(gf * gf, axis=1, keepdims=True)
    c2 = jnp.sum(cents * cents, axis=1)[None, :]
    cross = jax.lax.dot_general(gf, cents, (((1,), (1,)), ((), ())),
                                preferred_element_type=jnp.float32)
    d2 = x2 - 2.0 * cross + c2
    dmin = jnp.min(d2, axis=1, keepdims=True)
    m = (d2 <= dmin).astype(jnp.float32)
    ii = jax.lax.broadcasted_iota(jnp.int32, (_K, _K), 0)
    jj = jax.lax.broadcasted_iota(jnp.int32, (_K, _K), 1)
    lstrict = (ii < jj).astype(jnp.float32)
    prior = jnp.dot(m, lstrict, preferred_element_type=jnp.float32)
    onehot = jnp.where(prior == 0.0, m, 0.0)
    cluster = jnp.dot(onehot, cents, preferred_element_type=jnp.float32)

    xh = (jnp.dot(x_main, gw_ref[: _D], preferred_element_type=jnp.float32)
          + jnp.dot(cluster, gw_ref[_D:], preferred_element_type=jnp.float32)
          + gmb_ref[...])
    xh = _gelu_exact(xh)

    h = _layernorm(xh, l1g_ref[...], l1b_ref[...])
    h = jnp.dot(h, m1w_ref[...], preferred_element_type=jnp.float32) + m1b_ref[...]
    h = _gelu_exact(h)
    h = h + xh
    h = jnp.dot(h, m2w1_ref[...], preferred_element_type=jnp.float32) + m2b1_ref[...]
    h = _gelu_exact(h)
    h = jnp.dot(h, m2w2_ref[...], preferred_element_type=jnp.float32) + m2b2_ref[...]
    h = _gelu_exact(h)
    out_ref[...] = (jnp.dot(h, ow_ref[...], preferred_element_type=jnp.float32)
                    + ob_ref[...])


def kernel(x_embed, x_image, attn_w1, attn_b1, attn_w2, attn_b2,
           ln_ff_g, ln_ff_b, ff_w1, ff_b1, ff_w2, ff_b2,
           gram_w, gram_b, centers, gm_w, gm_b,
           ln1_g, ln1_b, m1_w, m1_b,
           m2_w1, m2_b1, m2_w2, m2_b2, out_w, out_b):
    xf = x_image.reshape(_B, _C, _HW)

    gram = pl.pallas_call(
        _gram_body,
        out_shape=jax.ShapeDtypeStruct((_B, _C, _C), jnp.bfloat16),
        grid=(_B // _GRAM_BB,),
        in_specs=[pl.BlockSpec((_GRAM_BB, _C, _HW), lambda i: (i, 0, 0))],
        out_specs=pl.BlockSpec((_GRAM_BB, _C, _C), lambda i: (i, 0, 0)),
        compiler_params=pltpu.CompilerParams(
            dimension_semantics=("parallel",),
            allow_input_fusion=(True,),
            vmem_limit_bytes=50 * 1024 * 1024,
        ),
        name="gram_bmm",
    )(xf)

    w3 = gram_w.reshape(_C, _C, _GR)

    full = lambda x: pl.BlockSpec(x.shape, lambda k: tuple(0 for _ in x.shape))
    head_args = (x_embed, gram_b, centers,
                 attn_w1, attn_b1, attn_w2, attn_b2,
                 ln_ff_g, ln_ff_b, ff_w1, ff_b1, ff_w2, ff_b2,
                 gm_w, gm_b, ln1_g, ln1_b, m1_w, m1_b,
                 m2_w1, m2_b1, m2_w2, m2_b2, out_w, out_b)
    logits = pl.pallas_call(
        _reduce_head_body,
        out_shape=jax.ShapeDtypeStruct((_B, _NC), jnp.float32),
        grid=(_RK,),
        in_specs=[
            pl.BlockSpec((_B, _RROWS, _C), lambda k: (0, k, 0)),
            pl.BlockSpec((_RROWS, _C, _GR), lambda k: (k, 0, 0)),
        ] + [full(a) for a in head_args],
        out_specs=pl.BlockSpec((_B, _NC), lambda k: (0, 0)),
        scratch_shapes=[pltpu.VMEM((_B, _GR), jnp.float32)],
        compiler_params=pltpu.CompilerParams(
            dimension_semantics=("arbitrary",),
            vmem_limit_bytes=56 * 1024 * 1024,
        ),
        name="gram_reduce_head",
    )(gram, w3, *head_args)
    return logits
